# Initial kernel scaffold; baseline (speedup 1.0000x reference)
import jax
import jax.numpy as jnp
from jax.experimental import pallas as pl


def kernel(y_true, y_pred, B):
    raise NotImplementedError("write your pallas kernel here")


# speedup vs baseline: 6.6083x; 6.6083x over previous
import jax
import jax.numpy as jnp
from jax.experimental import pallas as pl

_BLK = 16384


def _score_kernel(idx_ref, yp_ref, bt_ref, out_ref):
    step = pl.program_id(0)

    idx = idx_ref[...]
    yp = yp_ref[...]
    bt = bt_ref[...]

    iota = jax.lax.broadcasted_iota(jnp.int32, (_BLK, 24), 1)
    onehot = (idx == iota).astype(jnp.float32)
    gathered = jnp.dot(onehot, bt, preferred_element_type=jnp.float32)
    partial = jnp.sum(gathered * yp)

    @pl.when(step == 0)
    def _():
        out_ref[...] = jnp.zeros_like(out_ref)

    out_ref[...] = out_ref[...] + partial


def kernel(y_true, y_pred, B):
    n = y_true.shape[0] * y_true.shape[1]
    idx = y_true.reshape(n, 1).astype(jnp.int32)
    yp = y_pred.reshape(n, y_pred.shape[-1])
    bt = B.T

    grid = n // _BLK
    out = pl.pallas_call(
        _score_kernel,
        grid=(grid,),
        in_specs=[
            pl.BlockSpec((_BLK, 1), lambda i: (i, 0)),
            pl.BlockSpec((_BLK, 24), lambda i: (i, 0)),
            pl.BlockSpec((24, 24), lambda i: (0, 0)),
        ],
        out_specs=pl.BlockSpec((1, 1), lambda i: (0, 0)),
        out_shape=jax.ShapeDtypeStruct((1, 1), jnp.float32),
    )(idx, yp, bt)
    return out[0, 0]


# speedup vs baseline: 12.7101x; 1.9234x over previous
import jax
import jax.numpy as jnp
from jax.experimental import pallas as pl

_BLK = 32768


def _score_kernel(idx_ref, yp_ref, bt_ref, out_ref):
    step = pl.program_id(0)

    idx = idx_ref[...].reshape(1, _BLK)
    yp = yp_ref[...]
    bt = bt_ref[...]

    iota = jax.lax.broadcasted_iota(jnp.int32, (24, _BLK), 0)
    onehot = (idx == iota).astype(jnp.float32)
    s = jnp.dot(onehot, yp, preferred_element_type=jnp.float32)
    partial = jnp.sum(s * bt)

    @pl.when(step == 0)
    def _():
        out_ref[...] = jnp.zeros_like(out_ref)

    out_ref[...] = out_ref[...] + partial


def kernel(y_true, y_pred, B):
    n = y_true.shape[0] * y_true.shape[1]
    grid = n // _BLK
    idx = y_true.reshape(grid, 1, _BLK).astype(jnp.int32)
    yp = y_pred.reshape(n, y_pred.shape[-1])
    bt = B.T

    out = pl.pallas_call(
        _score_kernel,
        grid=(grid,),
        in_specs=[
            pl.BlockSpec((1, 1, _BLK), lambda i: (i, 0, 0)),
            pl.BlockSpec((_BLK, 24), lambda i: (i, 0)),
            pl.BlockSpec((24, 24), lambda i: (0, 0)),
        ],
        out_specs=pl.BlockSpec((1, 1), lambda i: (0, 0)),
        out_shape=jax.ShapeDtypeStruct((1, 1), jnp.float32),
    )(idx, yp, bt)
    return out[0, 0]
